# baseline (speedup 1.0000x reference)
import functools

import jax
import jax.numpy as jnp
from jax import lax
from jax.experimental import pallas as pl
from jax.experimental.pallas import tpu as pltpu
from jax.experimental.pallas import tpu_sc as plsc

_NUM_CORES = 2
_NUM_SUBCORES = 16
_NUM_WORKERS = _NUM_CORES * _NUM_SUBCORES


@functools.lru_cache(maxsize=None)
def _build_sc_embed(BS, NT, D, n_per_w):
    mesh = plsc.VectorSubcoreMesh(
        core_axis_name="c", subcore_axis_name="s", num_cores=_NUM_CORES,
        num_subcores=_NUM_SUBCORES)

    @functools.partial(
        pl.kernel,
        mesh=mesh,
        compiler_params=pltpu.CompilerParams(use_tc_tiling_on_sc=False),
        out_type=jax.ShapeDtypeStruct((BS, NT, D), jnp.float32),
        scratch_types=[
            pltpu.VMEM((n_per_w,), jnp.int32),
            pltpu.VMEM((n_per_w, D), jnp.float32),
            pltpu.SemaphoreType.DMA,
            pltpu.SemaphoreType.DMA,
        ],
    )
    def sc_embed(idx_hbm, *rest):
        tabs = rest[:NT]
        out_hbm, idx_v, rows_v, gsem, wsem = rest[NT:]
        wid = lax.axis_index("s") * _NUM_CORES + lax.axis_index("c")
        base = wid * n_per_w
        for v in range(NT):
            pltpu.sync_copy(idx_hbm.at[wid, v], idx_v)
            pltpu.async_copy(tabs[v].at[idx_v], rows_v, gsem).wait()
            pltpu.async_copy(
                rows_v, out_hbm.at[pl.ds(base, n_per_w), v], wsem).wait()

    return sc_embed


def kernel(x, tables):
    B, S, NX = x.shape
    D = tables[0].shape[1]
    NT = len(tables)
    BS = B * S
    LAG = tables[NX + 1].shape[0] - 1
    n_per_w = BS // _NUM_WORKERS

    x_t = jnp.transpose(x.astype(jnp.int32).reshape(BS, NX))
    s_row = jnp.tile(jnp.arange(S, dtype=jnp.int32), B)
    pf = jnp.concatenate(
        [jnp.zeros(S - LAG, jnp.int32), jnp.arange(1, LAG + 1, dtype=jnp.int32)])
    isf = jnp.concatenate(
        [jnp.zeros(S - LAG, jnp.int32), jnp.ones(LAG, jnp.int32)])
    idx_all = jnp.concatenate(
        [x_t, s_row[None], jnp.tile(pf, B)[None], jnp.tile(isf, B)[None]],
        axis=0)
    idx_w = jnp.transpose(
        idx_all.reshape(NT, _NUM_WORKERS, n_per_w), (1, 0, 2))

    out = _build_sc_embed(BS, NT, D, n_per_w)(idx_w, *tables)
    return out.reshape(B, S, NT, D)


# speedup vs baseline: 1.0612x; 1.0612x over previous
import functools

import jax
import jax.numpy as jnp
from jax import lax
from jax.experimental import pallas as pl
from jax.experimental.pallas import tpu as pltpu
from jax.experimental.pallas import tpu_sc as plsc

_NUM_CORES = 2
_NUM_SUBCORES = 16
_NUM_WORKERS = _NUM_CORES * _NUM_SUBCORES


@functools.lru_cache(maxsize=None)
def _build_sc_embed(BS, NT, D, n_per_w):
    mesh = plsc.VectorSubcoreMesh(
        core_axis_name="c", subcore_axis_name="s", num_cores=_NUM_CORES,
        num_subcores=_NUM_SUBCORES)

    @functools.partial(
        pl.kernel,
        mesh=mesh,
        compiler_params=pltpu.CompilerParams(use_tc_tiling_on_sc=False),
        out_type=jax.ShapeDtypeStruct((BS, NT, D), jnp.float32),
        scratch_types=[
            pltpu.VMEM((2, n_per_w), jnp.int32),
            pltpu.VMEM((2, n_per_w, D), jnp.float32),
            pltpu.SemaphoreType.DMA,
            pltpu.SemaphoreType.DMA,
            pltpu.SemaphoreType.DMA,
            pltpu.SemaphoreType.DMA,
        ],
    )
    def sc_embed(idx_hbm, *rest):
        tabs = rest[:NT]
        out_hbm, idx_v, rows_v, gs0, gs1, ws0, ws1 = rest[NT:]
        gsems = (gs0, gs1)
        wsems = (ws0, ws1)
        wid = lax.axis_index("s") * _NUM_CORES + lax.axis_index("c")
        base = wid * n_per_w

        gcps = [None] * NT
        wcps = [None] * NT

        def start_write(v):
            b = v & 1
            wcps[v] = pltpu.async_copy(
                rows_v.at[b], out_hbm.at[pl.ds(base, n_per_w), v], wsems[b])

        for v in range(NT):
            b = v & 1
            if v >= 2:
                wcps[v - 2].wait()
            pltpu.sync_copy(idx_hbm.at[v, pl.ds(base, n_per_w)], idx_v.at[b])
            gcps[v] = pltpu.async_copy(
                tabs[v].at[idx_v.at[b]], rows_v.at[b], gsems[b])
            if v >= 1:
                gcps[v - 1].wait()
                start_write(v - 1)
        gcps[NT - 1].wait()
        start_write(NT - 1)
        wcps[NT - 2].wait()
        wcps[NT - 1].wait()

    return sc_embed


def kernel(x, tables):
    B, S, NX = x.shape
    D = tables[0].shape[1]
    NT = len(tables)
    BS = B * S
    LAG = tables[NX + 1].shape[0] - 1
    n_per_w = BS // _NUM_WORKERS

    x_t = jnp.transpose(x.astype(jnp.int32).reshape(BS, NX))
    s_row = jnp.tile(jnp.arange(S, dtype=jnp.int32), B)
    pf = jnp.concatenate(
        [jnp.zeros(S - LAG, jnp.int32), jnp.arange(1, LAG + 1, dtype=jnp.int32)])
    isf = jnp.concatenate(
        [jnp.zeros(S - LAG, jnp.int32), jnp.ones(LAG, jnp.int32)])
    idx_all = jnp.concatenate(
        [x_t, s_row[None], jnp.tile(pf, B)[None], jnp.tile(isf, B)[None]],
        axis=0)

    out = _build_sc_embed(BS, NT, D, n_per_w)(idx_all, *tables)
    return out.reshape(B, S, NT, D)
